# Initial kernel scaffold; baseline (speedup 1.0000x reference)
import functools

import jax
import jax.numpy as jnp
from jax import lax
from jax.experimental import pallas as pl
from jax.experimental.pallas import tpu as pltpu
from jax.experimental.pallas import tpu_sc as plsc

_N = 10000
_E = 320000
_D = 128
_C = 64
_NC, _NS = 2, 16
_NW = _NC * _NS
_EPW = _E // _NW
_CHUNK = 80
_NCHUNK = _EPW // _CHUNK
_ROWS_PT = _N // _NS
_ZROWS = 25
_LANES = 16


def _agg_body(with_deg, x_hbm, src_hbm, dst_hbm, *refs):
    if with_deg:
        (out_hbm, deg_hbm, acc_sh, zbuf, src_v, dst_v, rows_v, deg_t,
         sem) = refs
    else:
        out_hbm, acc_sh, zbuf, src_v, dst_v, rows_v, sem = refs
    cid = lax.axis_index("c")
    sid = lax.axis_index("s")
    wid = cid * _NS + sid

    zeros16 = jnp.zeros((_LANES,), jnp.float32)
    ones16 = jnp.ones((_LANES,), jnp.float32)

    def _zrow(r, c):
        for k in range(_D // _LANES):
            zbuf[r, pl.ds(k * _LANES, _LANES)] = zeros16
        return c
    lax.fori_loop(0, _ZROWS, _zrow, 0)
    if with_deg:
        def _zdeg(i, c):
            deg_t[pl.ds(i * _LANES, _LANES)] = zeros16
            return c
        lax.fori_loop(0, _N // _LANES, _zdeg, 0)
    for j in range(_ROWS_PT // _ZROWS):
        pltpu.sync_copy(
            zbuf, acc_sh.at[pl.ds(sid * _ROWS_PT + j * _ZROWS, _ZROWS), :])
    plsc.subcore_barrier()

    def _chunk(i, c):
        base = wid * _EPW + i * _CHUNK
        pltpu.sync_copy(src_hbm.at[pl.ds(base, _CHUNK)], src_v)
        pltpu.sync_copy(dst_hbm.at[pl.ds(base, _CHUNK)], dst_v)
        pltpu.async_copy(x_hbm.at[src_v], rows_v, sem).wait()
        pltpu.sync_copy(rows_v, acc_sh.at[dst_v], add=True)
        if with_deg:
            for k in range(_CHUNK // _LANES):
                d16 = dst_v[pl.ds(k * _LANES, _LANES)]
                plsc.addupdate_scatter(deg_t, [d16], ones16)
        return c
    lax.fori_loop(0, _NCHUNK, _chunk, 0)
    plsc.subcore_barrier()

    pltpu.sync_copy(
        acc_sh.at[pl.ds(sid * _ROWS_PT, _ROWS_PT), :],
        out_hbm.at[cid, pl.ds(sid * _ROWS_PT, _ROWS_PT), :])
    if with_deg:
        pltpu.sync_copy(deg_t, deg_hbm.at[wid])


def _make_agg(with_deg):
    mesh = plsc.VectorSubcoreMesh(core_axis_name="c", subcore_axis_name="s")
    out_type = [jax.ShapeDtypeStruct((_NC, _N, _D), jnp.float32)]
    if with_deg:
        out_type.append(jax.ShapeDtypeStruct((_NW, _N), jnp.float32))
    scratch = [
        pltpu.VMEM_SHARED((_N, _D), jnp.float32),
        pltpu.VMEM((_ZROWS, _D), jnp.float32),
        pltpu.VMEM((_CHUNK,), jnp.int32),
        pltpu.VMEM((_CHUNK,), jnp.int32),
        pltpu.VMEM((_CHUNK, _D), jnp.float32),
    ]
    if with_deg:
        scratch.append(pltpu.VMEM((_N,), jnp.float32))
    scratch.append(pltpu.SemaphoreType.DMA)
    return pl.kernel(
        functools.partial(_agg_body, with_deg),
        out_type=out_type, mesh=mesh, scratch_types=scratch,
        name="sage_agg_deg" if with_deg else "sage_agg")


_agg_deg = _make_agg(True)
_agg = _make_agg(False)


def _layer1_body(x_ref, p_ref, degp_ref, wr_ref, wn_ref, b_ref, h1_ref):
    deg = jnp.sum(degp_ref[...], axis=1, keepdims=True)
    rdeg = 1.0 / jnp.maximum(deg, 1.0)
    mean = (p_ref[0] + p_ref[1]) * rdeg
    h = (jnp.dot(x_ref[...], wr_ref[...], preferred_element_type=jnp.float32)
         + jnp.dot(mean, wn_ref[...], preferred_element_type=jnp.float32)
         + b_ref[...])
    h1_ref[...] = jnp.maximum(h, 0.0)


def _layer2_body(h1_ref, p_ref, degp_ref, wr_ref, wn_ref, b_ref, wo_ref,
                 bo_ref, out_ref):
    deg = jnp.sum(degp_ref[...], axis=1, keepdims=True)
    rdeg = 1.0 / jnp.maximum(deg, 1.0)
    mean = (p_ref[0] + p_ref[1]) * rdeg
    h1 = h1_ref[...]
    h2 = (jnp.dot(h1, wr_ref[...], preferred_element_type=jnp.float32)
          + jnp.dot(mean, wn_ref[...], preferred_element_type=jnp.float32)
          + b_ref[...])
    h2 = jnp.maximum(h2, 0.0)
    wo = wo_ref[...]
    out_ref[...] = (
        jnp.dot(h1, wo[:_D], preferred_element_type=jnp.float32)
        + jnp.dot(h2, wo[_D:], preferred_element_type=jnp.float32)
        + bo_ref[...])


_R = 2000


def _tc_layer1(x, p, degp_t, W_root1, W_neigh1, b1):
    grid = (_N // _R,)
    return pl.pallas_call(
        _layer1_body,
        grid=grid,
        in_specs=[
            pl.BlockSpec((_R, _D), lambda i: (i, 0)),
            pl.BlockSpec((_NC, _R, _D), lambda i: (0, i, 0)),
            pl.BlockSpec((_R, _NW), lambda i: (i, 0)),
            pl.BlockSpec((_D, _D), lambda i: (0, 0)),
            pl.BlockSpec((_D, _D), lambda i: (0, 0)),
            pl.BlockSpec((1, _D), lambda i: (0, 0)),
        ],
        out_specs=pl.BlockSpec((_R, _D), lambda i: (i, 0)),
        out_shape=jax.ShapeDtypeStruct((_N, _D), jnp.float32),
        name="sage_tc1",
    )(x, p, degp_t, W_root1, W_neigh1, b1.reshape(1, _D))


def _tc_layer2(h1, p, degp_t, W_root2, W_neigh2, b2, W_out, b_out):
    grid = (_N // _R,)
    return pl.pallas_call(
        _layer2_body,
        grid=grid,
        in_specs=[
            pl.BlockSpec((_R, _D), lambda i: (i, 0)),
            pl.BlockSpec((_NC, _R, _D), lambda i: (0, i, 0)),
            pl.BlockSpec((_R, _NW), lambda i: (i, 0)),
            pl.BlockSpec((_D, _D), lambda i: (0, 0)),
            pl.BlockSpec((_D, _D), lambda i: (0, 0)),
            pl.BlockSpec((1, _D), lambda i: (0, 0)),
            pl.BlockSpec((2 * _D, _C), lambda i: (0, 0)),
            pl.BlockSpec((1, _C), lambda i: (0, 0)),
        ],
        out_specs=pl.BlockSpec((_R, _C), lambda i: (i, 0)),
        out_shape=jax.ShapeDtypeStruct((_N, _C), jnp.float32),
        name="sage_tc2",
    )(h1, p, degp_t, W_root2, W_neigh2, b2.reshape(1, _D), W_out,
      b_out.reshape(1, _C))


@jax.jit
def kernel(x, edge_index, W_root1, W_neigh1, b1, W_root2, W_neigh2, b2,
           W_out, b_out):
    src = edge_index[0]
    dst = edge_index[1]
    p1, degp = _agg_deg(x, src, dst)
    degp_t = degp.T
    h1 = _tc_layer1(x, p1, degp_t, W_root1, W_neigh1, b1)
    (p2,) = _agg(h1, src, dst)
    return _tc_layer2(h1, p2, degp_t, W_root2, W_neigh2, b2, W_out, b_out)


# speedup vs baseline: 5.5429x; 5.5429x over previous
import functools

import jax
import jax.numpy as jnp
from jax import lax
from jax.experimental import pallas as pl
from jax.experimental.pallas import tpu as pltpu
from jax.experimental.pallas import tpu_sc as plsc

_N = 10000
_E = 320000
_D = 128
_C = 64
_NC, _NS = 2, 16
_NW = _NC * _NS
_EPW = _E // _NW
_CHUNK = 80
_NCHUNK = _EPW // _CHUNK
_RCHUNK = 40
_NRCHUNK = _N // _RCHUNK
_LANES = 16


def _agg_body(with_deg, x_hbm, src_hbm, dst_hbm, *refs):
    if with_deg:
        (out_hbm, deg_hbm, acc_sh, zbuf, src_v, dst_v, rows_v, deg_t,
         sem) = refs
    else:
        out_hbm, acc_sh, zbuf, src_v, dst_v, rows_v, sem = refs
    cid = lax.axis_index("c")
    sid = lax.axis_index("s")
    wid = cid * _NS + sid

    zeros16 = jnp.zeros((_LANES,), jnp.float32)
    ones16 = jnp.ones((_LANES,), jnp.float32)

    def _zrow(r, c):
        for k in range(_D // _LANES):
            zbuf[r, pl.ds(k * _LANES, _LANES)] = zeros16
        return c
    lax.fori_loop(0, _RCHUNK, _zrow, 0)
    if with_deg:
        def _zdeg(i, c):
            deg_t[pl.ds(i * _LANES, _LANES)] = zeros16
            return c
        lax.fori_loop(0, _N // _LANES, _zdeg, 0)
    for j in range((_NRCHUNK + _NS - 1) // _NS):
        rc = sid + _NS * j
        @pl.when(rc < _NRCHUNK)
        def _():
            pltpu.sync_copy(zbuf, acc_sh.at[pl.ds(rc * _RCHUNK, _RCHUNK), :])
    plsc.subcore_barrier()

    def _chunk(i, c):
        base = wid * _EPW + i * _CHUNK
        pltpu.sync_copy(src_hbm.at[pl.ds(base, _CHUNK)], src_v)
        pltpu.sync_copy(dst_hbm.at[pl.ds(base, _CHUNK)], dst_v)
        pltpu.async_copy(x_hbm.at[src_v], rows_v, sem).wait()
        pltpu.sync_copy(rows_v, acc_sh.at[dst_v], add=True)
        if with_deg:
            for k in range(_CHUNK // _LANES):
                d16 = dst_v[pl.ds(k * _LANES, _LANES)]
                plsc.addupdate_scatter(deg_t, [d16], ones16)
        return c
    lax.fori_loop(0, _NCHUNK, _chunk, 0)
    plsc.subcore_barrier()

    for j in range((_NRCHUNK + _NS - 1) // _NS):
        rc = sid + _NS * j
        @pl.when(rc < _NRCHUNK)
        def _():
            pltpu.sync_copy(
                acc_sh.at[pl.ds(rc * _RCHUNK, _RCHUNK), :],
                out_hbm.at[cid, pl.ds(rc * _RCHUNK, _RCHUNK), :])
    if with_deg:
        pltpu.sync_copy(deg_t, deg_hbm.at[wid])


def _make_agg(with_deg):
    mesh = plsc.VectorSubcoreMesh(core_axis_name="c", subcore_axis_name="s")
    out_type = [jax.ShapeDtypeStruct((_NC, _N, _D), jnp.float32)]
    if with_deg:
        out_type.append(jax.ShapeDtypeStruct((_NW, _N), jnp.float32))
    scratch = [
        pltpu.VMEM_SHARED((_N, _D), jnp.float32),
        pltpu.VMEM((_RCHUNK, _D), jnp.float32),
        pltpu.VMEM((_CHUNK,), jnp.int32),
        pltpu.VMEM((_CHUNK,), jnp.int32),
        pltpu.VMEM((_CHUNK, _D), jnp.float32),
    ]
    if with_deg:
        scratch.append(pltpu.VMEM((_N,), jnp.float32))
    scratch.append(pltpu.SemaphoreType.DMA)
    return pl.kernel(
        functools.partial(_agg_body, with_deg),
        out_type=out_type, mesh=mesh, scratch_types=scratch,
        compiler_params=pltpu.CompilerParams(needs_layout_passes=False),
        name="sage_agg_deg" if with_deg else "sage_agg")


_agg_deg = _make_agg(True)
_agg = _make_agg(False)


def _layer1_body(x_ref, p_ref, degp_ref, wr_ref, wn_ref, b_ref, h1_ref):
    deg = jnp.sum(degp_ref[...], axis=1, keepdims=True)
    rdeg = 1.0 / jnp.maximum(deg, 1.0)
    mean = (p_ref[0] + p_ref[1]) * rdeg
    h = (jnp.dot(x_ref[...], wr_ref[...], preferred_element_type=jnp.float32)
         + jnp.dot(mean, wn_ref[...], preferred_element_type=jnp.float32)
         + b_ref[...])
    h1_ref[...] = jnp.maximum(h, 0.0)


def _layer2_body(h1_ref, p_ref, degp_ref, wr_ref, wn_ref, b_ref, wo_ref,
                 bo_ref, out_ref):
    deg = jnp.sum(degp_ref[...], axis=1, keepdims=True)
    rdeg = 1.0 / jnp.maximum(deg, 1.0)
    mean = (p_ref[0] + p_ref[1]) * rdeg
    h1 = h1_ref[...]
    h2 = (jnp.dot(h1, wr_ref[...], preferred_element_type=jnp.float32)
          + jnp.dot(mean, wn_ref[...], preferred_element_type=jnp.float32)
          + b_ref[...])
    h2 = jnp.maximum(h2, 0.0)
    wo = wo_ref[...]
    out_ref[...] = (
        jnp.dot(h1, wo[:_D], preferred_element_type=jnp.float32)
        + jnp.dot(h2, wo[_D:], preferred_element_type=jnp.float32)
        + bo_ref[...])


_R = 2000


def _tc_layer1(x, p, degp_t, W_root1, W_neigh1, b1):
    grid = (_N // _R,)
    return pl.pallas_call(
        _layer1_body,
        grid=grid,
        in_specs=[
            pl.BlockSpec((_R, _D), lambda i: (i, 0)),
            pl.BlockSpec((_NC, _R, _D), lambda i: (0, i, 0)),
            pl.BlockSpec((_R, _NW), lambda i: (i, 0)),
            pl.BlockSpec((_D, _D), lambda i: (0, 0)),
            pl.BlockSpec((_D, _D), lambda i: (0, 0)),
            pl.BlockSpec((1, _D), lambda i: (0, 0)),
        ],
        out_specs=pl.BlockSpec((_R, _D), lambda i: (i, 0)),
        out_shape=jax.ShapeDtypeStruct((_N, _D), jnp.float32),
        name="sage_tc1",
    )(x, p, degp_t, W_root1, W_neigh1, b1.reshape(1, _D))


def _tc_layer2(h1, p, degp_t, W_root2, W_neigh2, b2, W_out, b_out):
    grid = (_N // _R,)
    return pl.pallas_call(
        _layer2_body,
        grid=grid,
        in_specs=[
            pl.BlockSpec((_R, _D), lambda i: (i, 0)),
            pl.BlockSpec((_NC, _R, _D), lambda i: (0, i, 0)),
            pl.BlockSpec((_R, _NW), lambda i: (i, 0)),
            pl.BlockSpec((_D, _D), lambda i: (0, 0)),
            pl.BlockSpec((_D, _D), lambda i: (0, 0)),
            pl.BlockSpec((1, _D), lambda i: (0, 0)),
            pl.BlockSpec((2 * _D, _C), lambda i: (0, 0)),
            pl.BlockSpec((1, _C), lambda i: (0, 0)),
        ],
        out_specs=pl.BlockSpec((_R, _C), lambda i: (i, 0)),
        out_shape=jax.ShapeDtypeStruct((_N, _C), jnp.float32),
        name="sage_tc2",
    )(h1, p, degp_t, W_root2, W_neigh2, b2.reshape(1, _D), W_out,
      b_out.reshape(1, _C))


@jax.jit
def kernel(x, edge_index, W_root1, W_neigh1, b1, W_root2, W_neigh2, b2,
           W_out, b_out):
    src = edge_index[0]
    dst = edge_index[1]
    p1, degp = _agg_deg(x, src, dst)
    degp_t = degp.T
    h1 = _tc_layer1(x, p1, degp_t, W_root1, W_neigh1, b1)
    (p2,) = _agg(h1, src, dst)
    return _tc_layer2(h1, p2, degp_t, W_root2, W_neigh2, b2, W_out, b_out)
